# speedup vs baseline: 1.0882x; 1.0882x over previous
import functools

import jax
import jax.numpy as jnp
from jax import lax
from jax.experimental import pallas as pl
from jax.experimental.pallas import tpu as pltpu
from jax.experimental.pallas import tpu_sc as plsc

L_TX, L_SP, L_TP = 16, 8, 8
D = 768
B = 4096
TOK = L_TX + L_SP + L_TP
ROWS = B * TOK

NC, NS = 2, 16
NW = NC * NS
ROWS_PER_W = ROWS // NW
CHUNK = 128
NITER = ROWS_PER_W // CHUNK


@functools.partial(
    pl.kernel,
    out_type=jax.ShapeDtypeStruct((ROWS, D), jnp.float32),
    mesh=plsc.VectorSubcoreMesh(core_axis_name="c", subcore_axis_name="s"),
    scratch_types=[
        pltpu.VMEM((ROWS_PER_W,), jnp.int32),
        pltpu.VMEM((CHUNK, D), jnp.float32),
        pltpu.SemaphoreType.DMA,
    ],
)
def _gather_kernel(table_hbm, idx_hbm, out_hbm, idx_v, rows_v, sem):
    wid = lax.axis_index("s") * NC + lax.axis_index("c")
    base = wid * ROWS_PER_W
    pltpu.sync_copy(idx_hbm.at[pl.ds(base, ROWS_PER_W)], idx_v)

    def body(g, carry):
        off = g * CHUNK
        pltpu.async_copy(table_hbm.at[idx_v.at[pl.ds(off, CHUNK)]], rows_v, sem).wait()
        pltpu.sync_copy(rows_v, out_hbm.at[pl.ds(base + off, CHUNK)])
        return carry

    lax.fori_loop(0, NITER, body, 0)


def kernel(P_gn_txt, P_gn_ViT, P_gn_temp, idx_txt, idx_vit, idx_temp):
    table = jnp.concatenate([P_gn_txt, P_gn_ViT, P_gn_temp], axis=0)
    idx = jnp.concatenate(
        [idx_txt, idx_vit + L_TX, idx_temp + (L_TX + L_SP)], axis=1
    ).reshape(ROWS)
    out = _gather_kernel(table, idx)
    return out.reshape(B, TOK, D)


# speedup vs baseline: 1.1170x; 1.0264x over previous
import functools

import jax
import jax.numpy as jnp
from jax import lax
from jax.experimental import pallas as pl
from jax.experimental.pallas import tpu as pltpu
from jax.experimental.pallas import tpu_sc as plsc

L_TX, L_SP, L_TP = 16, 8, 8
D = 768
B = 4096
TOK = L_TX + L_SP + L_TP
ROWS = B * TOK

NC, NS = 2, 16
NW = NC * NS
ROWS_PER_W = ROWS // NW
CHUNK = 64
NITER = ROWS_PER_W // CHUNK
NHALF = NITER // 2


@functools.partial(
    pl.kernel,
    out_type=jax.ShapeDtypeStruct((ROWS, D), jnp.float32),
    mesh=plsc.VectorSubcoreMesh(core_axis_name="c", subcore_axis_name="s"),
    scratch_types=[
        pltpu.VMEM((ROWS_PER_W,), jnp.int32),
        pltpu.VMEM((CHUNK, D), jnp.float32),
        pltpu.VMEM((CHUNK, D), jnp.float32),
        pltpu.SemaphoreType.DMA,
        pltpu.SemaphoreType.DMA,
        pltpu.SemaphoreType.DMA,
        pltpu.SemaphoreType.DMA,
    ],
)
def _gather_kernel(table_hbm, idx_hbm, out_hbm, idx_v, rows0, rows1,
                   gs0, gs1, ss0, ss1):
    wid = lax.axis_index("s") * NC + lax.axis_index("c")
    base = wid * ROWS_PER_W
    pltpu.sync_copy(idx_hbm.at[pl.ds(base, ROWS_PER_W)], idx_v)

    rows = (rows0, rows1)
    gsem = (gs0, gs1)
    ssem = (ss0, ss1)

    def start_gather(b, g):
        off = g * CHUNK
        pltpu.async_copy(
            table_hbm.at[idx_v.at[pl.ds(off, CHUNK)]], rows[b], gsem[b])

    def wait_gather(b):
        pltpu.make_async_copy(table_hbm.at[idx_v.at[pl.ds(0, CHUNK)]],
                              rows[b], gsem[b]).wait()

    def start_scatter(b, g):
        off = g * CHUNK
        pltpu.async_copy(rows[b], out_hbm.at[pl.ds(base + off, CHUNK)], ssem[b])

    def wait_scatter(b):
        pltpu.make_async_copy(rows[b], out_hbm.at[pl.ds(base, CHUNK)],
                              ssem[b]).wait()

    start_gather(0, 0)

    def body(i, carry):
        g0 = i * 2
        wait_gather(0)
        start_scatter(0, g0)

        @pl.when(i > 0)
        def _():
            wait_scatter(1)
        start_gather(1, g0 + 1)

        wait_gather(1)
        start_scatter(1, g0 + 1)
        wait_scatter(0)

        @pl.when(i < NHALF - 1)
        def _():
            start_gather(0, g0 + 2)
        return carry

    lax.fori_loop(0, NHALF, body, 0)
    wait_scatter(1)


def kernel(P_gn_txt, P_gn_ViT, P_gn_temp, idx_txt, idx_vit, idx_temp):
    table = jnp.concatenate([P_gn_txt, P_gn_ViT, P_gn_temp], axis=0)
    idx = jnp.concatenate(
        [idx_txt, idx_vit + L_TX, idx_temp + (L_TX + L_SP)], axis=1
    ).reshape(ROWS)
    out = _gather_kernel(table, idx)
    return out.reshape(B, TOK, D)
